# baseline (speedup 1.0000x reference)
import functools

import jax
import jax.numpy as jnp
from jax import lax
from jax.experimental import pallas as pl
from jax.experimental.pallas import tpu as pltpu
from jax.experimental.pallas import tpu_sc as plsc

N = 10000
K = 32
C = 128
NC = 2
NS = 16
NW = NC * NS
NP = 10240
NPW = NP // NW
NLAST = N - (NW - 1) * NPW
CH = 4
NCHUNK = NPW // CH
NBUF = 4


def _sc_body(xt_hbm, adjf_hbm, out_hbm, table_sh, idx_all, rows, obuf, gsem, ssem):
    wid = lax.axis_index("s") * NC + lax.axis_index("c")
    base = wid * NPW
    full = base + NPW <= N
    @pl.when(lax.axis_index("s") == 0)
    def _():
        pltpu.sync_copy(xt_hbm, table_sh)

    @pl.when(full)
    def _():
        pltpu.sync_copy(adjf_hbm.at[pl.ds(base * K, NPW * K)], idx_all)

    @pl.when(jnp.logical_not(full))
    def _():
        pltpu.sync_copy(adjf_hbm.at[pl.ds(base * K, NLAST * K)],
                        idx_all.at[pl.ds(0, NLAST * K)])

    plsc.subcore_barrier()

    def _gather(c, b):
        return pltpu.make_async_copy(
            table_sh.at[idx_all.at[pl.ds(c * CH * K, CH * K)]], rows[b], gsem[b])

    def _store(c, b):
        return pltpu.make_async_copy(
            obuf[b], out_hbm.at[pl.ds(base + c * CH, CH)], ssem[b])

    nch = jnp.where(full, NCHUNK, NLAST // CH)

    for b in range(NBUF):
        _gather(b, b).start()

    @pl.loop(0, nch, step=NBUF)
    def _chunk(c):
        for b in range(NBUF):
            cc = c + b
            _gather(cc, b).wait()

            @pl.when(cc >= NBUF)
            def _():
                _store(cc - NBUF, b).wait()

            for i in range(CH):
                for g in range(C // 32):
                    v = rows[b][i * K, pl.ds(g * 16, 16)]
                    acc_e = lax.bitcast_convert_type(
                        lax.shift_left(v, jnp.int32(16)), jnp.float32)
                    acc_o = lax.bitcast_convert_type(v, jnp.float32)
                    for k in range(1, K):
                        v = rows[b][i * K + k, pl.ds(g * 16, 16)]
                        acc_e = acc_e + lax.bitcast_convert_type(
                            lax.shift_left(v, jnp.int32(16)), jnp.float32)
                        acc_o = acc_o + lax.bitcast_convert_type(v, jnp.float32)
                    obuf[b][i, pl.ds(g * 16, 16)] = acc_e
                    obuf[b][i, pl.ds(C // 2 + g * 16, 16)] = acc_o
            _store(cc, b).start()

            @pl.when(cc + NBUF < nch)
            def _():
                _gather(cc + NBUF, b).start()

    for b in range(NBUF):
        _store(nch - NBUF + b, b).wait()


_sc_gather_sum = functools.partial(
    pl.kernel,
    out_type=jax.ShapeDtypeStruct((NP, C), jnp.float32),
    mesh=plsc.VectorSubcoreMesh(core_axis_name="c", subcore_axis_name="s"),
    compiler_params=pltpu.CompilerParams(use_tc_tiling_on_sc=False),
    scratch_types=[
        pltpu.VMEM_SHARED((N, C // 2), jnp.int32),
        pltpu.VMEM((NPW * K,), jnp.int32),
        [pltpu.VMEM((CH * K, C // 2), jnp.int32) for _ in range(NBUF)],
        [pltpu.VMEM((CH, C), jnp.float32) for _ in range(NBUF)],
        [pltpu.SemaphoreType.DMA for _ in range(NBUF)],
        [pltpu.SemaphoreType.DMA for _ in range(NBUF)],
    ],
)(_sc_body)


NB = 1024


def _pack_body(x_ref, t_ref):
    xb = lax.bitcast_convert_type(x_ref[...], jnp.uint32)
    lo = xb[: C // 2, :]
    hi = xb[C // 2:, :]
    half = jnp.uint32(0x8000)
    ulo = lax.shift_right_logical(lo + half, jnp.uint32(16))
    uhi = lax.bitwise_and(hi + half, jnp.uint32(0xFFFF0000))
    w = lax.bitcast_convert_type(lax.bitwise_or(ulo, uhi), jnp.int32)
    t_ref[...] = w.T


def _pack_table(x):
    return pl.pallas_call(
        _pack_body,
        grid=(NP // NB,),
        in_specs=[pl.BlockSpec((C, NB), lambda i: (0, i))],
        out_specs=pl.BlockSpec((NB, C // 2), lambda i: (i, 0)),
        out_shape=jax.ShapeDtypeStruct((N, C // 2), jnp.int32),
    )(x)


def _y1_body(x_ref, w1_ref, b_ref, o_ref):
    o_ref[...] = jnp.dot(
        w1_ref[...], x_ref[...], preferred_element_type=jnp.float32
    ) + b_ref[...]


def _y1(x, w1, bias2d):
    return pl.pallas_call(
        _y1_body,
        grid=(NP // NB,),
        in_specs=[
            pl.BlockSpec((C, NB), lambda i: (0, i)),
            pl.BlockSpec((C, C), lambda i: (0, 0)),
            pl.BlockSpec((C, 1), lambda i: (0, 0)),
        ],
        out_specs=pl.BlockSpec((C, NB), lambda i: (0, i)),
        out_shape=jax.ShapeDtypeStruct((C, N), jnp.float32),
    )(x, w1, bias2d)


def _combine_body(y1_ref, g_ref, w2_ref, o_ref):
    o_ref[...] = y1_ref[...] + lax.dot_general(
        w2_ref[...], g_ref[...],
        (((1,), (1,)), ((), ())),
        preferred_element_type=jnp.float32,
    )


def _combine(y1, gsum, w2s):
    return pl.pallas_call(
        _combine_body,
        grid=(NP // NB,),
        in_specs=[
            pl.BlockSpec((C, NB), lambda i: (0, i)),
            pl.BlockSpec((NB, C), lambda i: (i, 0)),
            pl.BlockSpec((C, C), lambda i: (0, 0)),
        ],
        out_specs=pl.BlockSpec((C, NB), lambda i: (0, i)),
        out_shape=jax.ShapeDtypeStruct((C, N), jnp.float32),
    )(y1, gsum, w2s)


def kernel(input, adj, W1, b1, W2, b2):
    x = input.reshape(C, N)
    xt = _pack_table(x)
    adjf = adj.astype(jnp.int32).reshape(-1)
    gsum = _sc_gather_sum(xt, adjf)
    w2s = W2 * (1.0 / K)
    bias2d = (b1 + b2)[:, None]
    y1 = _y1(x, W1, bias2d)
    return _combine(y1, gsum, w2s).reshape(1, C, N)


# speedup vs baseline: 1.3503x; 1.3503x over previous
import functools

import jax
import jax.numpy as jnp
from jax import lax
from jax.experimental import pallas as pl
from jax.experimental.pallas import tpu as pltpu
from jax.experimental.pallas import tpu_sc as plsc

N = 10000
K = 32
C = 128
NC = 2
NS = 16
NW = NC * NS
NP = 10240
NPW = NP // NW
NLAST = N - (NW - 1) * NPW
CH = 4
NCHUNK = NPW // CH
NBUF = 2


def _sc_body(xt_hbm, adjf_hbm, out_hbm, table_sh, idx_all, rows, obuf, gsem, ssem):
    wid = lax.axis_index("s") * NC + lax.axis_index("c")
    base = wid * NPW
    full = base + NPW <= N
    @pl.when(lax.axis_index("s") == 0)
    def _():
        pltpu.sync_copy(xt_hbm, table_sh)

    @pl.when(full)
    def _():
        pltpu.sync_copy(adjf_hbm.at[pl.ds(base * K, NPW * K)], idx_all)

    @pl.when(jnp.logical_not(full))
    def _():
        pltpu.sync_copy(adjf_hbm.at[pl.ds(base * K, NLAST * K)],
                        idx_all.at[pl.ds(0, NLAST * K)])

    plsc.subcore_barrier()

    def _gather(c, b):
        return pltpu.make_async_copy(
            table_sh.at[idx_all.at[pl.ds(c * CH * K, CH * K)]], rows[b], gsem[b])

    def _store(c, b):
        return pltpu.make_async_copy(
            obuf[b], out_hbm.at[pl.ds(base + c * CH, CH)], ssem[b])

    nch = jnp.where(full, NCHUNK, NLAST // CH)

    for b in range(NBUF):
        _gather(b, b).start()

    @pl.loop(0, nch, step=NBUF)
    def _chunk(c):
        for b in range(NBUF):
            cc = c + b
            _gather(cc, b).wait()

            @pl.when(cc >= NBUF)
            def _():
                _store(cc - NBUF, b).wait()

            for i in range(CH):
                for g in range(C // 32):
                    v = rows[b][i * K, pl.ds(g * 16, 16)]
                    acc_e = lax.bitcast_convert_type(
                        lax.shift_left(v, jnp.int32(16)), jnp.float32)
                    acc_o = lax.bitcast_convert_type(v, jnp.float32)
                    for k in range(1, K):
                        v = rows[b][i * K + k, pl.ds(g * 16, 16)]
                        acc_e = acc_e + lax.bitcast_convert_type(
                            lax.shift_left(v, jnp.int32(16)), jnp.float32)
                        acc_o = acc_o + lax.bitcast_convert_type(v, jnp.float32)
                    obuf[b][i, pl.ds(g * 16, 16)] = acc_e
                    obuf[b][i, pl.ds(C // 2 + g * 16, 16)] = acc_o
            _store(cc, b).start()

            @pl.when(cc + NBUF < nch)
            def _():
                _gather(cc + NBUF, b).start()

    for b in range(NBUF):
        _store(nch - NBUF + b, b).wait()


_sc_gather_sum = functools.partial(
    pl.kernel,
    out_type=jax.ShapeDtypeStruct((NP, C), jnp.float32),
    mesh=plsc.VectorSubcoreMesh(core_axis_name="c", subcore_axis_name="s"),
    compiler_params=pltpu.CompilerParams(use_tc_tiling_on_sc=False),
    scratch_types=[
        pltpu.VMEM_SHARED((N, C // 2), jnp.int32),
        pltpu.VMEM((NPW * K,), jnp.int32),
        [pltpu.VMEM((CH * K, C // 2), jnp.int32) for _ in range(NBUF)],
        [pltpu.VMEM((CH, C), jnp.float32) for _ in range(NBUF)],
        [pltpu.SemaphoreType.DMA for _ in range(NBUF)],
        [pltpu.SemaphoreType.DMA for _ in range(NBUF)],
    ],
)(_sc_body)


NB = 1024


def _pack_body(x_ref, t_ref):
    xb = lax.bitcast_convert_type(x_ref[...], jnp.uint32)
    lo = xb[: C // 2, :]
    hi = xb[C // 2:, :]
    half = jnp.uint32(0x8000)
    ulo = lax.shift_right_logical(lo + half, jnp.uint32(16))
    uhi = lax.bitwise_and(hi + half, jnp.uint32(0xFFFF0000))
    w = lax.bitcast_convert_type(lax.bitwise_or(ulo, uhi), jnp.int32)
    t_ref[...] = w.T


def _pack_table(x):
    return pl.pallas_call(
        _pack_body,
        grid=(NP // NB,),
        in_specs=[pl.BlockSpec((C, NB), lambda i: (0, i))],
        out_specs=pl.BlockSpec((NB, C // 2), lambda i: (i, 0)),
        out_shape=jax.ShapeDtypeStruct((N, C // 2), jnp.int32),
    )(x)


def _y1_body(x_ref, w1_ref, b_ref, o_ref):
    o_ref[...] = jnp.dot(
        w1_ref[...], x_ref[...], preferred_element_type=jnp.float32
    ) + b_ref[...]


def _y1(x, w1, bias2d):
    return pl.pallas_call(
        _y1_body,
        grid=(NP // NB,),
        in_specs=[
            pl.BlockSpec((C, NB), lambda i: (0, i)),
            pl.BlockSpec((C, C), lambda i: (0, 0)),
            pl.BlockSpec((C, 1), lambda i: (0, 0)),
        ],
        out_specs=pl.BlockSpec((C, NB), lambda i: (0, i)),
        out_shape=jax.ShapeDtypeStruct((C, N), jnp.float32),
    )(x, w1, bias2d)


def _combine_body(y1_ref, g_ref, w2_ref, o_ref):
    o_ref[...] = y1_ref[...] + lax.dot_general(
        w2_ref[...], g_ref[...],
        (((1,), (1,)), ((), ())),
        preferred_element_type=jnp.float32,
    )


def _combine(y1, gsum, w2s):
    return pl.pallas_call(
        _combine_body,
        grid=(NP // NB,),
        in_specs=[
            pl.BlockSpec((C, NB), lambda i: (0, i)),
            pl.BlockSpec((NB, C), lambda i: (i, 0)),
            pl.BlockSpec((C, C), lambda i: (0, 0)),
        ],
        out_specs=pl.BlockSpec((C, NB), lambda i: (0, i)),
        out_shape=jax.ShapeDtypeStruct((C, N), jnp.float32),
    )(y1, gsum, w2s)


def kernel(input, adj, W1, b1, W2, b2):
    x = input.reshape(C, N)
    xt = _pack_table(x)
    adjf = adj.astype(jnp.int32).reshape(-1)
    gsum = _sc_gather_sum(xt, adjf)
    w2s = W2 * (1.0 / K)
    bias2d = (b1 + b2)[:, None]
    y1 = _y1(x, W1, bias2d)
    return _combine(y1, gsum, w2s).reshape(1, C, N)


# speedup vs baseline: 1.4114x; 1.0453x over previous
import functools

import jax
import jax.numpy as jnp
from jax import lax
from jax.experimental import pallas as pl
from jax.experimental.pallas import tpu as pltpu
from jax.experimental.pallas import tpu_sc as plsc

N = 10000
K = 32
C = 128
NC = 2
NS = 16
NW = NC * NS
NP = 10240
NPW = NP // NW
NLAST = N - (NW - 1) * NPW
CH = 4
NCHUNK = NPW // CH
NBUF = 2


def _sc_body(xt_hbm, adjf_hbm, out_hbm, table_sh, idx_all, rows, obuf, gsem, ssem):
    wid = lax.axis_index("s") * NC + lax.axis_index("c")
    base = wid * NPW
    full = base + NPW <= N
    @pl.when(lax.axis_index("s") == 0)
    def _():
        pltpu.sync_copy(xt_hbm, table_sh)

    @pl.when(full)
    def _():
        pltpu.sync_copy(adjf_hbm.at[pl.ds(base * K, NPW * K)], idx_all)

    @pl.when(jnp.logical_not(full))
    def _():
        pltpu.sync_copy(adjf_hbm.at[pl.ds(base * K, NLAST * K)],
                        idx_all.at[pl.ds(0, NLAST * K)])

    plsc.subcore_barrier()

    def _gather(c, b):
        return pltpu.make_async_copy(
            table_sh.at[idx_all.at[pl.ds(c * CH * K, CH * K)]], rows[b], gsem[b])

    def _store(c, b):
        return pltpu.make_async_copy(
            obuf[b], out_hbm.at[pl.ds(base + c * CH, CH)], ssem[b])

    nch = jnp.where(full, NCHUNK, NLAST // CH)

    for b in range(NBUF):
        _gather(b, b).start()

    @pl.loop(0, nch, step=NBUF)
    def _chunk(c):
        for b in range(NBUF):
            cc = c + b
            _gather(cc, b).wait()

            @pl.when(cc >= NBUF)
            def _():
                _store(cc - NBUF, b).wait()

            for i in range(CH):
                for g in range(C // 32):
                    v = rows[b][i * K, pl.ds(g * 16, 16)]
                    acc_e = lax.bitcast_convert_type(
                        lax.shift_left(v, jnp.int32(16)), jnp.float32)
                    acc_o = lax.bitcast_convert_type(v, jnp.float32)
                    for k in range(1, K):
                        v = rows[b][i * K + k, pl.ds(g * 16, 16)]
                        acc_e = acc_e + lax.bitcast_convert_type(
                            lax.shift_left(v, jnp.int32(16)), jnp.float32)
                        acc_o = acc_o + lax.bitcast_convert_type(v, jnp.float32)
                    obuf[b][i, pl.ds(g * 16, 16)] = acc_e
                    obuf[b][i, pl.ds(C // 2 + g * 16, 16)] = acc_o
            _store(cc, b).start()

            @pl.when(cc + NBUF < nch)
            def _():
                _gather(cc + NBUF, b).start()

    for b in range(NBUF):
        _store(nch - NBUF + b, b).wait()


_sc_gather_sum = functools.partial(
    pl.kernel,
    out_type=jax.ShapeDtypeStruct((NP, C), jnp.float32),
    mesh=plsc.VectorSubcoreMesh(core_axis_name="c", subcore_axis_name="s"),
    compiler_params=pltpu.CompilerParams(use_tc_tiling_on_sc=False),
    scratch_types=[
        pltpu.VMEM_SHARED((N, C // 2), jnp.int32),
        pltpu.VMEM((NPW * K,), jnp.int32),
        [pltpu.VMEM((CH * K, C // 2), jnp.int32) for _ in range(NBUF)],
        [pltpu.VMEM((CH, C), jnp.float32) for _ in range(NBUF)],
        [pltpu.SemaphoreType.DMA for _ in range(NBUF)],
        [pltpu.SemaphoreType.DMA for _ in range(NBUF)],
    ],
)(_sc_body)


NB = 2048


def _pack_body(x_ref, t_ref):
    xb = lax.bitcast_convert_type(x_ref[...], jnp.uint32)
    lo = xb[: C // 2, :]
    hi = xb[C // 2:, :]
    half = jnp.uint32(0x8000)
    ulo = lax.shift_right_logical(lo + half, jnp.uint32(16))
    uhi = lax.bitwise_and(hi + half, jnp.uint32(0xFFFF0000))
    w = lax.bitcast_convert_type(lax.bitwise_or(ulo, uhi), jnp.int32)
    t_ref[...] = w.T


def _pack_table(x):
    return pl.pallas_call(
        _pack_body,
        grid=(NP // NB,),
        in_specs=[pl.BlockSpec((C, NB), lambda i: (0, i))],
        out_specs=pl.BlockSpec((NB, C // 2), lambda i: (i, 0)),
        out_shape=jax.ShapeDtypeStruct((N, C // 2), jnp.int32),
    )(x)


def _combine_body(x_ref, g_ref, w1_ref, w2_ref, b_ref, o_ref):
    o_ref[...] = (
        jnp.dot(w1_ref[...], x_ref[...], preferred_element_type=jnp.float32)
        + lax.dot_general(
            w2_ref[...], g_ref[...],
            (((1,), (1,)), ((), ())),
            preferred_element_type=jnp.float32,
        )
        + b_ref[...]
    )


def _combine(x, gsum, w1, w2s, bias2d):
    return pl.pallas_call(
        _combine_body,
        grid=(NP // NB,),
        in_specs=[
            pl.BlockSpec((C, NB), lambda i: (0, i)),
            pl.BlockSpec((NB, C), lambda i: (i, 0)),
            pl.BlockSpec((C, C), lambda i: (0, 0)),
            pl.BlockSpec((C, C), lambda i: (0, 0)),
            pl.BlockSpec((C, 1), lambda i: (0, 0)),
        ],
        out_specs=pl.BlockSpec((C, NB), lambda i: (0, i)),
        out_shape=jax.ShapeDtypeStruct((C, N), jnp.float32),
    )(x, gsum, w1, w2s, bias2d)


def kernel(input, adj, W1, b1, W2, b2):
    x = input.reshape(C, N)
    xt = _pack_table(x)
    adjf = adj.astype(jnp.int32).reshape(-1)
    gsum = _sc_gather_sum(xt, adjf)
    w2s = W2 * (1.0 / K)
    bias2d = (b1 + b2)[:, None]
    return _combine(x, gsum, W1, w2s, bias2d).reshape(1, C, N)
